# baseline (speedup 1.0000x reference)
import functools

import jax
import jax.numpy as jnp
from jax import lax
from jax.experimental import pallas as pl
from jax.experimental.pallas import tpu as pltpu
from jax.experimental.pallas import tpu_sc as plsc

N = 10000
E = 320000
H = 512
CW = 128
NCH = H // CW
NSC = 2
NSUB = 16
EB = 128
E2 = 327680
NB = E2 // NSUB // EB
NGRP = 20
G = NB // NGRP
NB2 = E2 // (NSC * NSUB) // EB
NP = 10240
RPS = NP // NSUB
BM = 1000
MBLK = N // BM


@functools.cache
def _sc_kernels():
    mesh = plsc.VectorSubcoreMesh(core_axis_name="c", subcore_axis_name="s",
                                  num_cores=NSC, num_subcores=NSUB)

    @functools.partial(
        pl.kernel,
        mesh=mesh,
        out_type=jax.ShapeDtypeStruct((NCH, NP, CW), jnp.float32),
        scratch_types=[
            pltpu.VMEM((G, EB), jnp.int32),
            pltpu.VMEM((G, EB), jnp.int32),
            pltpu.VMEM((G, EB), jnp.int32),
            pltpu.VMEM((G, EB), jnp.int32),
            pltpu.VMEM((EB, CW), jnp.float32),
            pltpu.VMEM((EB, CW), jnp.float32),
            pltpu.VMEM_SHARED((NP, CW), jnp.float32),
            pltpu.SemaphoreType.DMA,
            pltpu.SemaphoreType.DMA,
            pltpu.SemaphoreType.DMA,
            pltpu.SemaphoreType.DMA,
            pltpu.SemaphoreType.DMA,
        ],
    )
    def scatter(src_hbm, dst_hbm, z_hbm, zstripe_hbm, out_hbm,
                srcA, dstA, srcB, dstB, b0, b1, acc,
                g00, g01, g10, g11, semi):
        cid = lax.axis_index("c")
        sid = lax.axis_index("s")
        bufs = (b0, b1)
        gsems = ((g00, g01), (g10, g11))
        HB = EB // 2

        def issue_gather(zc, s_idx, q, b):
            for h in range(2):
                pltpu.async_copy(zc.at[s_idx.at[q, pl.ds(h * HB, HB)]],
                                 bufs[b].at[pl.ds(h * HB, HB)], gsems[b][h])

        def wait_gather(zc, s_idx, q, b):
            for h in range(2):
                pltpu.make_async_copy(zc.at[s_idx.at[q, pl.ds(h * HB, HB)]],
                                      bufs[b].at[pl.ds(h * HB, HB)],
                                      gsems[b][h]).wait()

        row0 = sid * RPS

        def do_group(zc, s_idx, d_idx, ns_idx, ng, prime, wait_pref):
            for q in range(G):
                b = q % 2
                if q == G - 2 and wait_pref:
                    pltpu.make_async_copy(
                        src_hbm.at[sid].at[ng], ns_idx[0], semi).wait()
                    pltpu.make_async_copy(
                        dst_hbm.at[sid].at[ng], ns_idx[1], semi).wait()
                wait_gather(zc, s_idx, q, b)
                pltpu.sync_copy(bufs[b], acc.at[d_idx.at[q]], add=True)
                if q + 2 < G:
                    issue_gather(zc, s_idx, q + 2, b)
                elif prime:
                    issue_gather(zc, ns_idx[0], q + 2 - G, b)

        for c in range(NCH // NSC):
            chunk = cid * (NCH // NSC) + c
            zc = z_hbm.at[chunk]
            pltpu.sync_copy(zstripe_hbm, acc.at[pl.ds(row0, RPS)])
            pltpu.sync_copy(src_hbm.at[sid].at[0], srcA)
            pltpu.sync_copy(dst_hbm.at[sid].at[0], dstA)
            plsc.subcore_barrier()
            issue_gather(zc, srcA, 0, 0)
            issue_gather(zc, srcA, 1, 1)

            @pl.loop(0, NGRP, step=2)
            def _(g):
                pltpu.async_copy(src_hbm.at[sid].at[g + 1], srcB, semi)
                pltpu.async_copy(dst_hbm.at[sid].at[g + 1], dstB, semi)
                do_group(zc, srcA, dstA, (srcB, dstB), g + 1,
                         prime=True, wait_pref=True)

                @pl.when(g + 2 < NGRP)
                def _():
                    pltpu.async_copy(src_hbm.at[sid].at[g + 2], srcA, semi)
                    pltpu.async_copy(dst_hbm.at[sid].at[g + 2], dstA, semi)
                    do_group(zc, srcB, dstB, (srcA, dstA), g + 2,
                             prime=True, wait_pref=True)

                @pl.when(g + 2 >= NGRP)
                def _():
                    do_group(zc, srcB, dstB, (srcA, dstA), g + 2,
                             prime=False, wait_pref=False)

            plsc.subcore_barrier()
            pltpu.sync_copy(acc.at[pl.ds(row0, RPS)],
                            out_hbm.at[chunk].at[pl.ds(row0, RPS)])
            plsc.subcore_barrier()

    @functools.partial(
        pl.kernel,
        mesh=mesh,
        out_type=jax.ShapeDtypeStruct((NSC, NP, CW), jnp.float32),
        scratch_types=[
            pltpu.VMEM((NB2, EB), jnp.int32),
            pltpu.VMEM((EB, CW), jnp.float32),
            pltpu.VMEM_SHARED((NP, CW), jnp.float32),
        ],
    )
    def degree(dst_hbm, ones_hbm, zero_hbm, out_hbm,
               dst_v, ones_v, acc):
        cid = lax.axis_index("c")
        sid = lax.axis_index("s")
        pltpu.sync_copy(dst_hbm.at[cid].at[sid], dst_v)
        pltpu.sync_copy(ones_hbm, ones_v)
        row0 = sid * RPS
        pltpu.sync_copy(zero_hbm, acc.at[pl.ds(row0, RPS)])
        plsc.subcore_barrier()

        @pl.loop(0, NB2)
        def _(j):
            pltpu.sync_copy(ones_v, acc.at[dst_v.at[j]], add=True)

        plsc.subcore_barrier()
        pltpu.sync_copy(acc.at[pl.ds(row0, RPS)],
                        out_hbm.at[cid].at[pl.ds(row0, RPS)])

    return scatter, degree


def _sc_scatter(src, dst, z, zero_cw):
    return _sc_kernels()[0](src, dst, z, zero_cw)


def _sc_degree(dst, ones16, zero16):
    return _sc_kernels()[1](dst, ones16, zero16)


def _matmul(h, cs, cf, w, dis, bias, flat_out):
    kc = h.shape[0]

    def body(h_ref, cs_ref, cf_ref, w_ref, dis_ref, b_ref, o_ref):
        acc = jnp.zeros((BM, CW), jnp.float32)
        for c in range(kc):
            hn = h_ref[c] * cs_ref[c][None, :] + cf_ref[c][None, :]
            acc = acc + jnp.dot(hn, w_ref[c],
                                preferred_element_type=jnp.float32)
        z = acc * dis_ref[...] + b_ref[0]
        if flat_out:
            o_ref[...] = z
        else:
            o_ref[0] = z

    if flat_out:
        out_shape = jax.ShapeDtypeStruct((N, H), jnp.float32)
        out_spec = pl.BlockSpec((BM, CW), lambda i, j: (i, j))
    else:
        out_shape = jax.ShapeDtypeStruct((NCH, N, CW), jnp.float32)
        out_spec = pl.BlockSpec((1, BM, CW), lambda i, j: (j, i, 0))

    return pl.pallas_call(
        body,
        grid=(MBLK, NCH),
        in_specs=[
            pl.BlockSpec((kc, BM, CW), lambda i, j: (0, i, 0)),
            pl.BlockSpec((kc, CW), lambda i, j: (0, 0)),
            pl.BlockSpec((kc, CW), lambda i, j: (0, 0)),
            pl.BlockSpec((kc, CW, CW), lambda i, j: (0, 0, j)),
            pl.BlockSpec((BM, 1), lambda i, j: (i, 0)),
            pl.BlockSpec((1, 1, CW), lambda i, j: (j, 0, 0)),
        ],
        out_specs=out_spec,
        out_shape=out_shape,
    )(h, cs, cf, w, dis, bias)


def _combine(s, z, b, dis, leaky):
    def body(s_ref, z_ref, b_ref, dis_ref, a_ref, sums_ref, scr):
        i = pl.program_id(0)

        @pl.when(i == 0)
        def _():
            scr[...] = jnp.zeros_like(scr)

        d = dis_ref[...]
        for c in range(NCH):
            p = (s_ref[c] + z_ref[c]) * d + b_ref[c][None, :]
            if leaky:
                p = jnp.where(p > 0, p, 0.2 * p)
            a_ref[c] = p
            scr[0, c] += jnp.sum(p, axis=0)
            scr[1, c] += jnp.sum(p * p, axis=0)

        @pl.when(i == MBLK - 1)
        def _():
            sums_ref[...] = scr[...]

    return pl.pallas_call(
        body,
        grid=(MBLK,),
        in_specs=[
            pl.BlockSpec((NCH, BM, CW), lambda i: (0, i, 0)),
            pl.BlockSpec((NCH, BM, CW), lambda i: (0, i, 0)),
            pl.BlockSpec((NCH, CW), lambda i: (0, 0)),
            pl.BlockSpec((BM, 1), lambda i: (i, 0)),
        ],
        out_specs=[
            pl.BlockSpec((NCH, BM, CW), lambda i: (0, i, 0)),
            pl.BlockSpec((2, NCH, CW), lambda i: (0, 0, 0)),
        ],
        out_shape=[
            jax.ShapeDtypeStruct((NCH, N, CW), jnp.float32),
            jax.ShapeDtypeStruct((2, NCH, CW), jnp.float32),
        ],
        scratch_shapes=[pltpu.VMEM((2, NCH, CW), jnp.float32)],
    )(s, z, b, dis)


def kernel(x, edge_index, edge_attr,
           W0, b0, g0, be0, W1, b1, g1, be1, W2, b2, g2, be2,
           W3, b3, g3, be3, W4, b4, g4, be4, Wf, bf):
    del edge_attr
    order = jnp.argsort(edge_index[0])
    src_s = jnp.take(edge_index[0], order)
    dst_s = jnp.take(edge_index[1], order)
    npad = E2 - E
    srcp = jnp.concatenate([src_s, jnp.zeros((npad,), edge_index.dtype)])
    dstp = jnp.concatenate([dst_s, jnp.full((npad,), N, edge_index.dtype)])
    src4 = srcp.reshape(NSUB, NGRP, G, EB)
    dst4 = dstp.reshape(NSUB, NGRP, G, EB)
    dst3 = dstp.reshape(NSC, NSUB, NB2, EB)

    zstripe = jnp.zeros((RPS, CW), jnp.float32)
    ones_cw = jnp.ones((EB, CW), jnp.float32)

    deg2 = _sc_degree(dst3, ones_cw, zstripe)
    deg = deg2[0, :N, 0:1] + deg2[1, :N, 0:1]
    dis = lax.rsqrt(deg + 1.0)

    Ws = [W0.reshape(1, CW, H)] + [W.reshape(NCH, CW, H)
                                   for W in (W1, W2, W3, W4)]
    bs = [b.reshape(NCH, CW) for b in (b0, b1, b2, b3, b4)]
    gs = [g.reshape(NCH, CW) for g in (g0, g1, g2, g3, g4)]
    bes = [be.reshape(NCH, CW) for be in (be0, be1, be2, be3, be4)]
    zero_bias = jnp.zeros((NCH, 1, CW), jnp.float32)

    h = x.reshape(1, N, CW)
    cs = jnp.ones((1, CW), jnp.float32)
    cf = jnp.zeros((1, CW), jnp.float32)

    for i in range(5):
        z = _matmul(h, cs, cf, Ws[i], dis, zero_bias, flat_out=False)
        s = _sc_scatter(src4, dst4, z, zstripe)
        h, sums = _combine(s, z, bs[i], dis, leaky=(i < 4))
        m = sums[0] / N
        var = sums[1] / N - m * m
        inv = lax.rsqrt(var + 1e-5)
        cs = gs[i] * inv
        cf = bes[i] - m * cs

    ones_dis = jnp.ones((N, 1), jnp.float32)
    wft = jnp.transpose(Wf).reshape(NCH, CW, H)
    return _matmul(h, cs, cf, wft, ones_dis, bf.reshape(NCH, 1, CW),
                   flat_out=True)


# speedup vs baseline: 1.2298x; 1.2298x over previous
import functools

import jax
import jax.numpy as jnp
from jax import lax
from jax.experimental import pallas as pl
from jax.experimental.pallas import tpu as pltpu
from jax.experimental.pallas import tpu_sc as plsc

N = 10000
E = 320000
H = 512
CW = 128
NCH = H // CW
NSC = 2
NSUB = 16
EB = 128
E2 = 327680
NB = E2 // NSUB // EB
NGRP = 20
G = NB // NGRP
NB2 = E2 // (NSC * NSUB) // EB
NP = 10240
RPS = NP // NSUB
BM = 1000
MBLK = N // BM


@functools.cache
def _sc_kernels():
    mesh = plsc.VectorSubcoreMesh(core_axis_name="c", subcore_axis_name="s",
                                  num_cores=NSC, num_subcores=NSUB)

    @functools.partial(
        pl.kernel,
        mesh=mesh,
        out_type=jax.ShapeDtypeStruct((NCH, NP, CW), jnp.float32),
        scratch_types=[
            pltpu.VMEM((G, EB), jnp.int32),
            pltpu.VMEM((G, EB), jnp.int32),
            pltpu.VMEM((G, EB), jnp.int32),
            pltpu.VMEM((G, EB), jnp.int32),
            pltpu.VMEM((EB, CW), jnp.float32),
            pltpu.VMEM((EB, CW), jnp.float32),
            pltpu.VMEM_SHARED((NP, CW), jnp.float32),
            pltpu.SemaphoreType.DMA,
            pltpu.SemaphoreType.DMA,
            pltpu.SemaphoreType.DMA,
            pltpu.SemaphoreType.DMA,
            pltpu.SemaphoreType.DMA,
        ],
    )
    def scatter(src_hbm, dst_hbm, z_hbm, zstripe_hbm, out_hbm,
                srcA, dstA, srcB, dstB, b0, b1, acc,
                g00, g01, g10, g11, semi):
        cid = lax.axis_index("c")
        sid = lax.axis_index("s")
        bufs = (b0, b1)
        gsems = ((g00, g01), (g10, g11))
        HB = EB // 2

        def issue_gather(zc, s_idx, q, b):
            for h in range(2):
                pltpu.async_copy(zc.at[s_idx.at[q, pl.ds(h * HB, HB)]],
                                 bufs[b].at[pl.ds(h * HB, HB)], gsems[b][h])

        def wait_gather(zc, s_idx, q, b):
            for h in range(2):
                pltpu.make_async_copy(zc.at[s_idx.at[q, pl.ds(h * HB, HB)]],
                                      bufs[b].at[pl.ds(h * HB, HB)],
                                      gsems[b][h]).wait()

        row0 = sid * RPS

        def do_group(zc, s_idx, d_idx, ns_idx, ng, prime, wait_pref):
            for q in range(G):
                b = q % 2
                if q == G - 2 and wait_pref:
                    pltpu.make_async_copy(
                        src_hbm.at[sid].at[ng], ns_idx[0], semi).wait()
                    pltpu.make_async_copy(
                        dst_hbm.at[sid].at[ng], ns_idx[1], semi).wait()
                wait_gather(zc, s_idx, q, b)
                pltpu.sync_copy(bufs[b], acc.at[d_idx.at[q]], add=True)
                if q + 2 < G:
                    issue_gather(zc, s_idx, q + 2, b)
                elif prime:
                    issue_gather(zc, ns_idx[0], q + 2 - G, b)

        for c in range(NCH // NSC):
            chunk = cid * (NCH // NSC) + c
            zc = z_hbm.at[chunk]
            pltpu.sync_copy(zstripe_hbm, acc.at[pl.ds(row0, RPS)])
            pltpu.sync_copy(src_hbm.at[sid].at[0], srcA)
            pltpu.sync_copy(dst_hbm.at[sid].at[0], dstA)
            plsc.subcore_barrier()
            issue_gather(zc, srcA, 0, 0)
            issue_gather(zc, srcA, 1, 1)

            @pl.loop(0, NGRP, step=2)
            def _(g):
                pltpu.async_copy(src_hbm.at[sid].at[g + 1], srcB, semi)
                pltpu.async_copy(dst_hbm.at[sid].at[g + 1], dstB, semi)
                do_group(zc, srcA, dstA, (srcB, dstB), g + 1,
                         prime=True, wait_pref=True)

                @pl.when(g + 2 < NGRP)
                def _():
                    pltpu.async_copy(src_hbm.at[sid].at[g + 2], srcA, semi)
                    pltpu.async_copy(dst_hbm.at[sid].at[g + 2], dstA, semi)
                    do_group(zc, srcB, dstB, (srcA, dstA), g + 2,
                             prime=True, wait_pref=True)

                @pl.when(g + 2 >= NGRP)
                def _():
                    do_group(zc, srcB, dstB, (srcA, dstA), g + 2,
                             prime=False, wait_pref=False)

            plsc.subcore_barrier()
            pltpu.sync_copy(acc.at[pl.ds(row0, RPS)],
                            out_hbm.at[chunk].at[pl.ds(row0, RPS)])
            plsc.subcore_barrier()

    @functools.partial(
        pl.kernel,
        mesh=mesh,
        out_type=jax.ShapeDtypeStruct((NSC, NP, CW), jnp.float32),
        scratch_types=[
            pltpu.VMEM((NB2, EB), jnp.int32),
            pltpu.VMEM((EB, CW), jnp.float32),
            pltpu.VMEM_SHARED((NP, CW), jnp.float32),
        ],
    )
    def degree(dst_hbm, ones_hbm, zero_hbm, out_hbm,
               dst_v, ones_v, acc):
        cid = lax.axis_index("c")
        sid = lax.axis_index("s")
        pltpu.sync_copy(dst_hbm.at[cid].at[sid], dst_v)
        pltpu.sync_copy(ones_hbm, ones_v)
        row0 = sid * RPS
        pltpu.sync_copy(zero_hbm, acc.at[pl.ds(row0, RPS)])
        plsc.subcore_barrier()

        @pl.loop(0, NB2)
        def _(j):
            pltpu.sync_copy(ones_v, acc.at[dst_v.at[j]], add=True)

        plsc.subcore_barrier()
        pltpu.sync_copy(acc.at[pl.ds(row0, RPS)],
                        out_hbm.at[cid].at[pl.ds(row0, RPS)])

    return scatter, degree


def _sc_scatter(src, dst, z, zero_cw):
    return _sc_kernels()[0](src, dst, z, zero_cw)


def _sc_degree(dst, ones16, zero16):
    return _sc_kernels()[1](dst, ones16, zero16)


def _matmul(h, cs, cf, w, dis, bias, flat_out):
    kc = h.shape[0]

    def body(h_ref, cs_ref, cf_ref, w_ref, dis_ref, b_ref, o_ref):
        acc = jnp.zeros((BM, CW), jnp.float32)
        for c in range(kc):
            hn = h_ref[c] * cs_ref[c][None, :] + cf_ref[c][None, :]
            acc = acc + jnp.dot(hn, w_ref[c],
                                preferred_element_type=jnp.float32)
        z = acc * dis_ref[...] + b_ref[0]
        if flat_out:
            o_ref[...] = z
        else:
            o_ref[0] = z

    if flat_out:
        out_shape = jax.ShapeDtypeStruct((N, H), jnp.float32)
        out_spec = pl.BlockSpec((BM, CW), lambda i, j: (i, j))
    else:
        out_shape = jax.ShapeDtypeStruct((NCH, N, CW), jnp.float32)
        out_spec = pl.BlockSpec((1, BM, CW), lambda i, j: (j, i, 0))

    return pl.pallas_call(
        body,
        grid=(MBLK, NCH),
        in_specs=[
            pl.BlockSpec((kc, BM, CW), lambda i, j: (0, i, 0)),
            pl.BlockSpec((kc, CW), lambda i, j: (0, 0)),
            pl.BlockSpec((kc, CW), lambda i, j: (0, 0)),
            pl.BlockSpec((kc, CW, CW), lambda i, j: (0, 0, j)),
            pl.BlockSpec((BM, 1), lambda i, j: (i, 0)),
            pl.BlockSpec((1, 1, CW), lambda i, j: (j, 0, 0)),
        ],
        out_specs=out_spec,
        out_shape=out_shape,
    )(h, cs, cf, w, dis, bias)


def _combine(s, z, b, dis, leaky):
    def body(s_ref, z_ref, b_ref, dis_ref, a_ref, sums_ref, scr):
        i = pl.program_id(0)

        @pl.when(i == 0)
        def _():
            scr[...] = jnp.zeros_like(scr)

        d = dis_ref[...]
        for c in range(NCH):
            p = (s_ref[c] + z_ref[c]) * d + b_ref[c][None, :]
            if leaky:
                p = jnp.where(p > 0, p, 0.2 * p)
            a_ref[c] = p
            scr[0, c] += jnp.sum(p, axis=0)
            scr[1, c] += jnp.sum(p * p, axis=0)

        @pl.when(i == MBLK - 1)
        def _():
            sums_ref[...] = scr[...]

    return pl.pallas_call(
        body,
        grid=(MBLK,),
        in_specs=[
            pl.BlockSpec((NCH, BM, CW), lambda i: (0, i, 0)),
            pl.BlockSpec((NCH, BM, CW), lambda i: (0, i, 0)),
            pl.BlockSpec((NCH, CW), lambda i: (0, 0)),
            pl.BlockSpec((BM, 1), lambda i: (i, 0)),
        ],
        out_specs=[
            pl.BlockSpec((NCH, BM, CW), lambda i: (0, i, 0)),
            pl.BlockSpec((2, NCH, CW), lambda i: (0, 0, 0)),
        ],
        out_shape=[
            jax.ShapeDtypeStruct((NCH, N, CW), jnp.float32),
            jax.ShapeDtypeStruct((2, NCH, CW), jnp.float32),
        ],
        scratch_shapes=[pltpu.VMEM((2, NCH, CW), jnp.float32)],
    )(s, z, b, dis)


def kernel(x, edge_index, edge_attr,
           W0, b0, g0, be0, W1, b1, g1, be1, W2, b2, g2, be2,
           W3, b3, g3, be3, W4, b4, g4, be4, Wf, bf):
    del edge_attr
    npad = E2 - E
    srcp = jnp.concatenate(
        [edge_index[0], jnp.zeros((npad,), edge_index.dtype)])
    dstp = jnp.concatenate(
        [edge_index[1], jnp.full((npad,), N, edge_index.dtype)])
    src4 = srcp.reshape(NSUB, NGRP, G, EB)
    dst4 = dstp.reshape(NSUB, NGRP, G, EB)
    dst3 = dstp.reshape(NSC, NSUB, NB2, EB)

    zstripe = jnp.zeros((RPS, CW), jnp.float32)
    ones_cw = jnp.ones((EB, CW), jnp.float32)

    deg2 = _sc_degree(dst3, ones_cw, zstripe)
    deg = deg2[0, :N, 0:1] + deg2[1, :N, 0:1]
    dis = lax.rsqrt(deg + 1.0)

    Ws = [W0.reshape(1, CW, H)] + [W.reshape(NCH, CW, H)
                                   for W in (W1, W2, W3, W4)]
    bs = [b.reshape(NCH, CW) for b in (b0, b1, b2, b3, b4)]
    gs = [g.reshape(NCH, CW) for g in (g0, g1, g2, g3, g4)]
    bes = [be.reshape(NCH, CW) for be in (be0, be1, be2, be3, be4)]
    zero_bias = jnp.zeros((NCH, 1, CW), jnp.float32)

    h = x.reshape(1, N, CW)
    cs = jnp.ones((1, CW), jnp.float32)
    cf = jnp.zeros((1, CW), jnp.float32)

    for i in range(5):
        z = _matmul(h, cs, cf, Ws[i], dis, zero_bias, flat_out=False)
        s = _sc_scatter(src4, dst4, z, zstripe)
        h, sums = _combine(s, z, bs[i], dis, leaky=(i < 4))
        m = sums[0] / N
        var = sums[1] / N - m * m
        inv = lax.rsqrt(var + 1e-5)
        cs = gs[i] * inv
        cf = bes[i] - m * cs

    ones_dis = jnp.ones((N, 1), jnp.float32)
    wft = jnp.transpose(Wf).reshape(NCH, CW, H)
    return _matmul(h, cs, cf, wft, ones_dis, bf.reshape(NCH, 1, CW),
                   flat_out=True)
